# baseline (speedup 1.0000x reference)
import functools

import jax
import jax.numpy as jnp
from jax import lax
from jax.experimental import pallas as pl
from jax.experimental.pallas import tpu as pltpu
from jax.experimental.pallas import tpu_sc as plsc

SEQ = 200
BATCH = 1024
D = 128
B = SEQ * BATCH

NC = 2
NS = 16
NW = NC * NS
L = 16

C = 128
CHUNKS = B // C
CHUNKS_PER_W = CHUNKS // NW
CHUNKS_PER_SEQ = BATCH // C


def _embed_kernel(x_hbm, pe_hbm, table_hbm, out_hbm, idx_v, rows_v, pe_v, sem):
    wid = lax.axis_index("s") * NC + lax.axis_index("c")

    def chunk_body(i, carry):
        g = wid * CHUNKS_PER_W + i
        s = g // CHUNKS_PER_SEQ
        pltpu.sync_copy(x_hbm.at[pl.ds(g * C, C)], idx_v)
        pltpu.sync_copy(pe_hbm.at[pl.ds(s * D, D)], pe_v)
        pltpu.async_copy(table_hbm.at[idx_v], rows_v, sem).wait()
        pe_regs = [pe_v[pl.ds(j * L, L)] for j in range(D // L)]

        def row_body(r, c2):
            for j in range(D // L):
                sl = pl.ds(j * L, L)
                rows_v[r, sl] = rows_v[r, sl] + pe_regs[j]
            return c2

        lax.fori_loop(0, C, row_body, 0)
        pltpu.sync_copy(rows_v, out_hbm.at[pl.ds(g * C, C)])
        return carry

    lax.fori_loop(0, CHUNKS_PER_W, chunk_body, 0)


@jax.jit
def _embed(x_flat, pe_flat, table):
    mesh = plsc.VectorSubcoreMesh(core_axis_name="c", subcore_axis_name="s")
    kern = functools.partial(
        pl.kernel,
        out_type=jax.ShapeDtypeStruct((B, D), jnp.float32),
        mesh=mesh,
        scratch_types=[
            pltpu.VMEM((C,), jnp.int32),
            pltpu.VMEM((C, D), jnp.float32),
            pltpu.VMEM((D,), jnp.float32),
            pltpu.SemaphoreType.DMA,
        ],
    )(_embed_kernel)
    return kern(x_flat, pe_flat, table)


def kernel(x, table, pe):
    x_flat = x.reshape(-1).astype(jnp.int32)
    pe_flat = pe.reshape(-1)
    out = _embed(x_flat, pe_flat, table)
    return out.reshape(SEQ, BATCH, D)
